# baseline (speedup 1.0000x reference)
import functools

import jax
import jax.numpy as jnp
from jax.experimental import pallas as pl
from jax.experimental.pallas import tpu as pltpu


def _mask_cols(x1tt, j, bj, e):
    col = jax.lax.broadcasted_iota(jnp.int32, (1, x1tt.shape[1]), 1) + j * bj
    return jnp.where(col < e, x1tt, 0.0)


def _contract_last(a, b):
    return jax.lax.dot_general(a, b, (((1,), (1,)), ((), ())),
                               preferred_element_type=jnp.float32)


def _body1(x0t_ref, h_ref, w0t_ref, x0at_ref, h8_ref, acc_ref, *, nj, bj, e):
    j = pl.program_id(0)

    stripe = h_ref[...].astype(jnp.bfloat16)
    h8_ref[...] = stripe.astype(jnp.int8)

    x1tt = jnp.dot(x0t_ref[...], stripe,
                   preferred_element_type=jnp.float32)
    if e % bj != 0:
        x1tt = _mask_cols(x1tt, j, bj, e)

    yt = jnp.dot(w0t_ref[...].astype(jnp.bfloat16), x1tt.astype(jnp.bfloat16),
                 preferred_element_type=jnp.float32)
    contrib = _contract_last(stripe, yt.astype(jnp.bfloat16))

    @pl.when(j == 0)
    def _():
        acc_ref[...] = contrib

    @pl.when(j > 0)
    def _():
        acc_ref[...] += contrib

    @pl.when(j == nj - 1)
    def _():
        x0at_ref[...] = acc_ref[...].T.astype(jnp.bfloat16)


def _body2(x0at_ref, h8_ref, w1t_ref, x0_out_ref, x1_out_ref, *, nj, bj, e):
    j = pl.program_id(0)

    stripe = h8_ref[...].astype(jnp.bfloat16)

    x1tt = jnp.dot(x0at_ref[...], stripe,
                   preferred_element_type=jnp.float32)
    if e % bj != 0:
        x1tt = _mask_cols(x1tt, j, bj, e)
    x1_out_ref[...] = x1tt.T

    yt = jnp.dot(w1t_ref[...].astype(jnp.bfloat16), x1tt.astype(jnp.bfloat16),
                 preferred_element_type=jnp.float32)
    contrib = _contract_last(stripe, yt.astype(jnp.bfloat16))

    @pl.when(j == 0)
    def _():
        x0_out_ref[...] = contrib

    @pl.when(j > 0)
    def _():
        x0_out_ref[...] += contrib


def kernel(x_0, incidence_1, W0, W1):
    n, c = x_0.shape
    e = incidence_1.shape[1]
    bj1 = 256
    nj1 = -(-e // bj1)
    bj2 = 512
    nj2 = -(-e // bj2)

    x0t = x_0.T.astype(jnp.bfloat16)
    w0t = W0.T
    w1t = W1.T

    x0at, h8 = pl.pallas_call(
        functools.partial(_body1, nj=nj1, bj=bj1, e=e),
        grid=(nj1,),
        in_specs=[
            pl.BlockSpec((c, n), lambda j: (0, 0)),
            pl.BlockSpec((n, bj1), lambda j: (0, j)),
            pl.BlockSpec((c, c), lambda j: (0, 0)),
        ],
        out_specs=[
            pl.BlockSpec((c, n), lambda j: (0, 0)),
            pl.BlockSpec((n, bj1), lambda j: (0, j)),
        ],
        out_shape=(
            jax.ShapeDtypeStruct((c, n), jnp.bfloat16),
            jax.ShapeDtypeStruct((n, e), jnp.int8),
        ),
        scratch_shapes=[
            pltpu.VMEM((n, c), jnp.float32),
        ],
        compiler_params=pltpu.CompilerParams(
            dimension_semantics=("arbitrary",)),
    )(x0t, incidence_1, w0t)

    x0_out, x1_out = pl.pallas_call(
        functools.partial(_body2, nj=nj2, bj=bj2, e=e),
        grid=(nj2,),
        in_specs=[
            pl.BlockSpec((c, n), lambda j: (0, 0)),
            pl.BlockSpec((n, bj2), lambda j: (0, j)),
            pl.BlockSpec((c, c), lambda j: (0, 0)),
        ],
        out_specs=[
            pl.BlockSpec((n, c), lambda j: (0, 0)),
            pl.BlockSpec((bj2, c), lambda j: (j, 0)),
        ],
        out_shape=(
            jax.ShapeDtypeStruct((n, c), jnp.float32),
            jax.ShapeDtypeStruct((e, c), jnp.float32),
        ),
        compiler_params=pltpu.CompilerParams(
            dimension_semantics=("arbitrary",)),
    )(x0at, h8, w1t)
    return x0_out, x1_out


# speedup vs baseline: 1.1768x; 1.1768x over previous
import functools

import jax
import jax.numpy as jnp
from jax.experimental import pallas as pl
from jax.experimental.pallas import tpu as pltpu


def _body(x0t_ref, h_ref, w0_ref, w1_ref, x0_out_ref, x1_out_ref,
          *, nj, bj, e):
    l = pl.program_id(0)
    j = pl.program_id(1)

    stripe = h_ref[...].astype(jnp.bfloat16)

    x1tt = jnp.dot(x0t_ref[...], stripe,
                   preferred_element_type=jnp.float32)
    x1t = x1tt.T
    if e % bj != 0:
        row = jax.lax.broadcasted_iota(jnp.int32, (bj, 1), 0) + j * bj
        x1t = jnp.where(row < e, x1t, 0.0)
    x1_out_ref[...] = x1t

    w = jnp.where(l == 0, w0_ref[...], w1_ref[...]).astype(jnp.bfloat16)
    y = jnp.dot(x1t.astype(jnp.bfloat16), w,
                preferred_element_type=jnp.float32)

    contrib = jnp.dot(stripe, y.astype(jnp.bfloat16),
                      preferred_element_type=jnp.float32)

    @pl.when(j == 0)
    def _():
        x0_out_ref[...] = contrib

    @pl.when(j > 0)
    def _():
        x0_out_ref[...] += contrib

    @pl.when(jnp.logical_and(l == 0, j == nj - 1))
    def _():
        x0t_ref[...] = x0_out_ref[...].T.astype(jnp.bfloat16)


def kernel(x_0, incidence_1, W0, W1):
    n, c = x_0.shape
    e = incidence_1.shape[1]
    bj = 512
    nj = -(-e // bj)

    x0t = x_0.T.astype(jnp.bfloat16)

    grid = (2, nj)
    out_shape = (
        jax.ShapeDtypeStruct((n, c), jnp.float32),
        jax.ShapeDtypeStruct((e, c), jnp.float32),
    )
    x0_out, x1_out = pl.pallas_call(
        functools.partial(_body, nj=nj, bj=bj, e=e),
        grid=grid,
        in_specs=[
            pl.BlockSpec((c, n), lambda l, j: (0, 0)),
            pl.BlockSpec((n, bj), lambda l, j: (0, j)),
            pl.BlockSpec((c, c), lambda l, j: (0, 0)),
            pl.BlockSpec((c, c), lambda l, j: (0, 0)),
        ],
        out_specs=[
            pl.BlockSpec((n, c), lambda l, j: (0, 0)),
            pl.BlockSpec((bj, c), lambda l, j: (j, 0)),
        ],
        out_shape=out_shape,
        compiler_params=pltpu.CompilerParams(
            dimension_semantics=("arbitrary", "arbitrary")),
    )(x0t, incidence_1, W0, W1)
    return x0_out, x1_out
